# baseline (speedup 1.0000x reference)
import functools

import jax
import jax.numpy as jnp
from jax import lax
from jax.experimental import pallas as pl
from jax.experimental.pallas import tpu as pltpu
from jax.experimental.pallas import tpu_sc as plsc

_N = 10000
_NP = 10240
_K = 16
_PROP = 22
_HC = 32
_BR = 128
_BE = 1024
_BM = 512
_NW = 32
_NPW = _NP // _NW
_NB = 16
_EWL = 16


def _elu(v):
    return jnp.where(v > 0, v, jnp.exp(v) - 1.0)


def _enc_kernel(x_ref, w0, b0, w1, b1, w2, b2, ws, bs, wh, bh,
                x1_ref, s_ref, h_ref):
    x = x_ref[...]
    t = _elu(jnp.dot(x, w0[...], preferred_element_type=jnp.float32) + b0[...])
    t = _elu(jnp.dot(t, w1[...], preferred_element_type=jnp.float32) + b1[...])
    x1 = jnp.dot(t, w2[...], preferred_element_type=jnp.float32) + b2[...]
    x1_ref[...] = x1
    s_ref[...] = jnp.dot(x1, ws[...], preferred_element_type=jnp.float32) + bs[...]
    h_ref[...] = jnp.dot(x1, wh[...], preferred_element_type=jnp.float32) + bh[...]


def _knn_kernel(s_ref, st_ref, idx_ref, ew_ref):
    s_r = s_ref[...]
    st = st_ref[...]
    sq_r = jnp.sum(s_r * s_r, axis=1, keepdims=True)
    sq_c = jnp.sum(st * st, axis=0, keepdims=True)
    m = lax.dot_general(s_r, st, (((1,), (0,)), ((), ())),
                        preferred_element_type=jnp.float32)
    d2 = sq_r + sq_c - 2.0 * m
    col = lax.broadcasted_iota(jnp.int32, (_BR, _NP), 1)
    d2 = jnp.where(col >= _N, jnp.inf, d2)
    for k in range(_K):
        mv = jnp.min(d2, axis=1, keepdims=True)
        sel = jnp.min(jnp.where(d2 == mv, col, _NP), axis=1, keepdims=True)
        idx_ref[:, k:k + 1] = sel
        wk = jnp.exp(-10.0 * jnp.maximum(mv, 0.0))
        ew_ref[:, k * _EWL:(k + 1) * _EWL] = jnp.broadcast_to(wk, (_BR, _EWL))
        d2 = jnp.where(col == sel, jnp.inf, d2)


def _agg_body(h_hbm, idx_hbm, ewx_hbm, sum_hbm, max_hbm,
              idx_v, ewx_v, rows_v, sum_v, max_v, sem):
    wid = lax.axis_index("s") * 2 + lax.axis_index("c")
    node0 = wid * _NPW

    def block(b, carry):
        nb = node0 + b * _NB
        e0 = pl.multiple_of(nb * _K, _NB * _K)
        pltpu.sync_copy(idx_hbm.at[pl.ds(e0, _NB * _K)], idx_v)
        pltpu.sync_copy(ewx_hbm.at[pl.ds(nb, _NB)], ewx_v)
        pltpu.async_copy(h_hbm.at[idx_v], rows_v, sem).wait()
        for nn in range(_NB):
            for ch in range(0, _HC, 16):
                accm = jnp.zeros((16,), jnp.float32)
                accx = jnp.full((16,), -jnp.inf, jnp.float32)
                for k in range(_K):
                    v = rows_v[_K * nn + k, ch:ch + 16]
                    w = ewx_v[nn, _EWL * k:_EWL * k + 16]
                    msg = v * w
                    accm = accm + msg
                    accx = jnp.maximum(accx, msg)
                sum_v[nn, ch:ch + 16] = accm
                max_v[nn, ch:ch + 16] = accx
        pltpu.sync_copy(sum_v, sum_hbm.at[pl.ds(nb, _NB)])
        pltpu.sync_copy(max_v, max_hbm.at[pl.ds(nb, _NB)])
        return carry

    lax.fori_loop(0, _NPW // _NB, block, 0)


def _mlp_kernel(x1_ref, x_ref, sum_ref, max_ref,
                wox, wom, woM, bo,
                n2w0, n2b0, n2w1, n2b1, n2w2, n2b2, n2w3, n2b3,
                n3wx, n3wi, n3wc, n3b0, n3w1, n3b1, n3w2, n3b2, n3w3, n3b3,
                ids_ref, p4_ref):
    xc = (jnp.dot(x1_ref[...], wox[...], preferred_element_type=jnp.float32)
          + jnp.dot(sum_ref[...], wom[...], preferred_element_type=jnp.float32)
          + jnp.dot(max_ref[...], woM[...], preferred_element_type=jnp.float32)
          + bo[...])
    xc = jnp.where(xc > 0, xc, 0.01 * xc)

    t = _elu(jnp.dot(xc, n2w0[...], preferred_element_type=jnp.float32) + n2b0[...])
    t = _elu(jnp.dot(t, n2w1[...], preferred_element_type=jnp.float32) + n2b1[...])
    t = _elu(jnp.dot(t, n2w2[...], preferred_element_type=jnp.float32) + n2b2[...])
    ids = jnp.dot(t, n2w3[...], preferred_element_type=jnp.float32) + n2b3[...]
    ids_ref[...] = ids

    u = (jnp.dot(x_ref[...], n3wx[...], preferred_element_type=jnp.float32)
         + jnp.dot(ids, n3wi[...], preferred_element_type=jnp.float32)
         + jnp.dot(xc, n3wc[...], preferred_element_type=jnp.float32)
         + n3b0[...])
    u = _elu(u)
    u = _elu(jnp.dot(u, n3w1[...], preferred_element_type=jnp.float32) + n3b1[...])
    u = _elu(jnp.dot(u, n3w2[...], preferred_element_type=jnp.float32) + n3b2[...])
    p4_ref[...] = jnp.dot(u, n3w3[...], preferred_element_type=jnp.float32) + n3b3[...]


def _full(shape):
    nd = len(shape)
    return pl.BlockSpec(shape, lambda i, _nd=nd: (0,) * _nd)


def kernel(x, ygen_id, ygen, ycand_id, ycand, params):
    p = params
    f32 = jnp.float32
    xp = jnp.pad(x, ((0, _NP - _N), (0, 0)))

    def b2d(name):
        return p[name].reshape(1, -1)

    wh32 = jnp.pad(p['gn_Wh'], ((0, 0), (0, _HC - _PROP)))
    bh32 = jnp.pad(p['gn_bh'], ((0, _HC - _PROP))).reshape(1, -1)

    x1, s, h32 = pl.pallas_call(
        _enc_kernel,
        grid=(_NP // _BE,),
        in_specs=[
            pl.BlockSpec((_BE, 12), lambda i: (i, 0)),
            _full((12, 64)), _full((1, 64)),
            _full((64, 64)), _full((1, 64)),
            _full((64, 12)), _full((1, 12)),
            _full((12, 4)), _full((1, 4)),
            _full((12, _HC)), _full((1, _HC)),
        ],
        out_specs=[
            pl.BlockSpec((_BE, 12), lambda i: (i, 0)),
            pl.BlockSpec((_BE, 4), lambda i: (i, 0)),
            pl.BlockSpec((_BE, _HC), lambda i: (i, 0)),
        ],
        out_shape=[
            jax.ShapeDtypeStruct((_NP, 12), f32),
            jax.ShapeDtypeStruct((_NP, 4), f32),
            jax.ShapeDtypeStruct((_NP, _HC), f32),
        ],
        compiler_params=pltpu.CompilerParams(
            dimension_semantics=("arbitrary",)),
    )(xp, p['nn1_W0'], b2d('nn1_b0'), p['nn1_W1'], b2d('nn1_b1'),
      p['nn1_W2'], b2d('nn1_b2'), p['gn_Ws'], b2d('gn_bs'), wh32, bh32)

    st = s.T

    idx, ew = pl.pallas_call(
        _knn_kernel,
        grid=(_NP // _BR,),
        in_specs=[
            pl.BlockSpec((_BR, 4), lambda i: (i, 0)),
            _full((4, _NP)),
        ],
        out_specs=[
            pl.BlockSpec((_BR, _K), lambda i: (i, 0)),
            pl.BlockSpec((_BR, _K * _EWL), lambda i: (i, 0)),
        ],
        out_shape=[
            jax.ShapeDtypeStruct((_NP, _K), jnp.int32),
            jax.ShapeDtypeStruct((_NP, _K * _EWL), f32),
        ],
        compiler_params=pltpu.CompilerParams(
            dimension_semantics=("arbitrary",)),
    )(s, st)

    idxf = idx.reshape(_NP * _K)

    mesh = plsc.VectorSubcoreMesh(core_axis_name="c", subcore_axis_name="s")
    sum32, max32 = pl.kernel(
        _agg_body,
        out_type=[jax.ShapeDtypeStruct((_NP, _HC), f32),
                  jax.ShapeDtypeStruct((_NP, _HC), f32)],
        mesh=mesh,
        scratch_types=[
            pltpu.VMEM((_NB * _K,), jnp.int32),
            pltpu.VMEM((_NB, _K * _EWL), f32),
            pltpu.VMEM((_NB * _K, _HC), f32),
            pltpu.VMEM((_NB, _HC), f32),
            pltpu.VMEM((_NB, _HC), f32),
            pltpu.SemaphoreType.DMA,
        ],
        compiler_params=pltpu.CompilerParams(use_tc_tiling_on_sc=False),
    )(h32, idxf, ew)

    wo = p['gn_Wo']
    wox = wo[:12]
    wom32 = jnp.pad(wo[12:12 + _PROP], ((0, _HC - _PROP), (0, 0))) * (1.0 / _K)
    woM32 = jnp.pad(wo[12 + _PROP:], ((0, _HC - _PROP), (0, 0)))
    n3w0 = p['nn3_W0']
    n3wx, n3wi, n3wc = n3w0[:12], n3w0[12:18], n3w0[18:]

    ids, p4 = pl.pallas_call(
        _mlp_kernel,
        grid=(_NP // _BM,),
        in_specs=[
            pl.BlockSpec((_BM, 12), lambda i: (i, 0)),
            pl.BlockSpec((_BM, 12), lambda i: (i, 0)),
            pl.BlockSpec((_BM, _HC), lambda i: (i, 0)),
            pl.BlockSpec((_BM, _HC), lambda i: (i, 0)),
            _full((12, 64)), _full((_HC, 64)), _full((_HC, 64)), _full((1, 64)),
            _full((64, 256)), _full((1, 256)),
            _full((256, 256)), _full((1, 256)),
            _full((256, 256)), _full((1, 256)),
            _full((256, 6)), _full((1, 6)),
            _full((12, 256)), _full((6, 256)), _full((64, 256)), _full((1, 256)),
            _full((256, 256)), _full((1, 256)),
            _full((256, 256)), _full((1, 256)),
            _full((256, 6)), _full((1, 6)),
        ],
        out_specs=[
            pl.BlockSpec((_BM, 6), lambda i: (i, 0)),
            pl.BlockSpec((_BM, 6), lambda i: (i, 0)),
        ],
        out_shape=[
            jax.ShapeDtypeStruct((_NP, 6), f32),
            jax.ShapeDtypeStruct((_NP, 6), f32),
        ],
        compiler_params=pltpu.CompilerParams(
            dimension_semantics=("arbitrary",)),
    )(x1, xp, sum32, max32,
      wox, wom32, woM32, b2d('gn_bo'),
      p['nn2_W0'], b2d('nn2_b0'), p['nn2_W1'], b2d('nn2_b1'),
      p['nn2_W2'], b2d('nn2_b2'), p['nn2_W3'], b2d('nn2_b3'),
      n3wx, n3wi, n3wc, b2d('nn3_b0'),
      p['nn3_W1'], b2d('nn3_b1'), p['nn3_W2'], b2d('nn3_b2'),
      p['nn3_W3'], b2d('nn3_b3'))

    return (ids[:_N], p4[:_N], ygen_id, ygen, ycand_id, ycand)


# speedup vs baseline: 1.4951x; 1.4951x over previous
import functools

import jax
import jax.numpy as jnp
import numpy as _np
from jax import lax
from jax.experimental import pallas as pl
from jax.experimental.pallas import tpu as pltpu
from jax.experimental.pallas import tpu_sc as plsc

_N = 10000
_NP = 10240
_K = 16
_PROP = 22
_HC = 32
_BR = 128
_BE = 1024
_BM = 512
_NW = 32
_NPW = _NP // _NW
_NB = 16
_EWL = 16


def _elu(v):
    return jnp.where(v > 0, v, jnp.exp(v) - 1.0)


def _enc_kernel(x_ref, w0, b0, w1, b1, w2, b2, ws, bs, wh, bh,
                x1_ref, s_ref, h_ref):
    x = x_ref[...]
    t = _elu(jnp.dot(x, w0[...], preferred_element_type=jnp.float32) + b0[...])
    t = _elu(jnp.dot(t, w1[...], preferred_element_type=jnp.float32) + b1[...])
    x1 = jnp.dot(t, w2[...], preferred_element_type=jnp.float32) + b2[...]
    x1_ref[...] = x1
    s_ref[...] = jnp.dot(x1, ws[...], preferred_element_type=jnp.float32) + bs[...]
    h_ref[...] = jnp.dot(x1, wh[...], preferred_element_type=jnp.float32) + bh[...]


_NQ = 4
_QW = _NP // _NQ
_BIGQ = [float(_np.frombuffer(_np.uint32(0x7F000000 + _i).tobytes(),
                              _np.float32)[0]) for _i in range(_NQ)]


def _knn_kernel(s_ref, st_ref, idx_ref, ew_ref):
    s_r = s_ref[...]
    st = st_ref[...]
    sq_r = jnp.sum(s_r * s_r, axis=1, keepdims=True)
    sq_c = jnp.sum(st * st, axis=0, keepdims=True)
    m = lax.dot_general(s_r, st, (((1,), (0,)), ((), ())),
                        preferred_element_type=jnp.float32)
    d2 = sq_r + sq_c - 2.0 * m
    colq = lax.broadcasted_iota(jnp.int32, (_BR, _QW), 1)
    q = []
    for i in range(_NQ):
        di = d2[:, i * _QW:(i + 1) * _QW]
        if i == _NQ - 1:
            di = jnp.where(colq >= _N - (_NQ - 1) * _QW, _BIGQ[i], di)
        bi = (lax.bitcast_convert_type(di, jnp.int32) & ~3) | i
        q.append(lax.bitcast_convert_type(bi, jnp.float32))
    for k in range(_K):
        dm = jnp.minimum(jnp.minimum(q[0], q[1]), jnp.minimum(q[2], q[3]))
        mv = jnp.min(dm, axis=1, keepdims=True)
        selq = jnp.argmin(dm, axis=1, keepdims=True).astype(jnp.int32)
        qi = lax.bitcast_convert_type(mv, jnp.int32) & 3
        sel = selq + _QW * qi
        idx_ref[:, k:k + 1] = sel
        wk = jnp.exp(-10.0 * jnp.maximum(mv, 0.0))
        ew_ref[:, k * _EWL:(k + 1) * _EWL] = jnp.broadcast_to(wk, (_BR, _EWL))
        for i in range(_NQ):
            q[i] = jnp.where(colq == sel - i * _QW, _BIGQ[i], q[i])


def _agg_body(h_hbm, idx_hbm, ewx_hbm, sum_hbm, max_hbm,
              idx_v, ewx_v, rows_v, sum_v, max_v, sem):
    wid = lax.axis_index("s") * 2 + lax.axis_index("c")
    node0 = wid * _NPW

    def block(b, carry):
        nb = node0 + b * _NB
        e0 = pl.multiple_of(nb * _K, _NB * _K)
        pltpu.sync_copy(idx_hbm.at[pl.ds(e0, _NB * _K)], idx_v)
        pltpu.sync_copy(ewx_hbm.at[pl.ds(nb, _NB)], ewx_v)
        pltpu.async_copy(h_hbm.at[idx_v], rows_v, sem).wait()
        for nn in range(_NB):
            for ch in range(0, _HC, 16):
                accm = jnp.zeros((16,), jnp.float32)
                accx = jnp.full((16,), -jnp.inf, jnp.float32)
                for k in range(_K):
                    v = rows_v[_K * nn + k, ch:ch + 16]
                    w = ewx_v[nn, _EWL * k:_EWL * k + 16]
                    msg = v * w
                    accm = accm + msg
                    accx = jnp.maximum(accx, msg)
                sum_v[nn, ch:ch + 16] = accm
                max_v[nn, ch:ch + 16] = accx
        pltpu.sync_copy(sum_v, sum_hbm.at[pl.ds(nb, _NB)])
        pltpu.sync_copy(max_v, max_hbm.at[pl.ds(nb, _NB)])
        return carry

    lax.fori_loop(0, _NPW // _NB, block, 0)


def _mlp_kernel(x1_ref, x_ref, sum_ref, max_ref,
                wox, wom, woM, bo,
                n2w0, n2b0, n2w1, n2b1, n2w2, n2b2, n2w3, n2b3,
                n3wx, n3wi, n3wc, n3b0, n3w1, n3b1, n3w2, n3b2, n3w3, n3b3,
                ids_ref, p4_ref):
    xc = (jnp.dot(x1_ref[...], wox[...], preferred_element_type=jnp.float32)
          + jnp.dot(sum_ref[...], wom[...], preferred_element_type=jnp.float32)
          + jnp.dot(max_ref[...], woM[...], preferred_element_type=jnp.float32)
          + bo[...])
    xc = jnp.where(xc > 0, xc, 0.01 * xc)

    t = _elu(jnp.dot(xc, n2w0[...], preferred_element_type=jnp.float32) + n2b0[...])
    t = _elu(jnp.dot(t, n2w1[...], preferred_element_type=jnp.float32) + n2b1[...])
    t = _elu(jnp.dot(t, n2w2[...], preferred_element_type=jnp.float32) + n2b2[...])
    ids = jnp.dot(t, n2w3[...], preferred_element_type=jnp.float32) + n2b3[...]
    ids_ref[...] = ids

    u = (jnp.dot(x_ref[...], n3wx[...], preferred_element_type=jnp.float32)
         + jnp.dot(ids, n3wi[...], preferred_element_type=jnp.float32)
         + jnp.dot(xc, n3wc[...], preferred_element_type=jnp.float32)
         + n3b0[...])
    u = _elu(u)
    u = _elu(jnp.dot(u, n3w1[...], preferred_element_type=jnp.float32) + n3b1[...])
    u = _elu(jnp.dot(u, n3w2[...], preferred_element_type=jnp.float32) + n3b2[...])
    p4_ref[...] = jnp.dot(u, n3w3[...], preferred_element_type=jnp.float32) + n3b3[...]


def _full(shape):
    nd = len(shape)
    return pl.BlockSpec(shape, lambda i, _nd=nd: (0,) * _nd)


def kernel(x, ygen_id, ygen, ycand_id, ycand, params):
    p = params
    f32 = jnp.float32
    xp = jnp.pad(x, ((0, _NP - _N), (0, 0)))

    def b2d(name):
        return p[name].reshape(1, -1)

    wh32 = jnp.pad(p['gn_Wh'], ((0, 0), (0, _HC - _PROP)))
    bh32 = jnp.pad(p['gn_bh'], ((0, _HC - _PROP))).reshape(1, -1)

    x1, s, h32 = pl.pallas_call(
        _enc_kernel,
        grid=(_NP // _BE,),
        in_specs=[
            pl.BlockSpec((_BE, 12), lambda i: (i, 0)),
            _full((12, 64)), _full((1, 64)),
            _full((64, 64)), _full((1, 64)),
            _full((64, 12)), _full((1, 12)),
            _full((12, 4)), _full((1, 4)),
            _full((12, _HC)), _full((1, _HC)),
        ],
        out_specs=[
            pl.BlockSpec((_BE, 12), lambda i: (i, 0)),
            pl.BlockSpec((_BE, 4), lambda i: (i, 0)),
            pl.BlockSpec((_BE, _HC), lambda i: (i, 0)),
        ],
        out_shape=[
            jax.ShapeDtypeStruct((_NP, 12), f32),
            jax.ShapeDtypeStruct((_NP, 4), f32),
            jax.ShapeDtypeStruct((_NP, _HC), f32),
        ],
        compiler_params=pltpu.CompilerParams(
            dimension_semantics=("arbitrary",)),
    )(xp, p['nn1_W0'], b2d('nn1_b0'), p['nn1_W1'], b2d('nn1_b1'),
      p['nn1_W2'], b2d('nn1_b2'), p['gn_Ws'], b2d('gn_bs'), wh32, bh32)

    st = s.T

    idx, ew = pl.pallas_call(
        _knn_kernel,
        grid=(_NP // _BR,),
        in_specs=[
            pl.BlockSpec((_BR, 4), lambda i: (i, 0)),
            _full((4, _NP)),
        ],
        out_specs=[
            pl.BlockSpec((_BR, _K), lambda i: (i, 0)),
            pl.BlockSpec((_BR, _K * _EWL), lambda i: (i, 0)),
        ],
        out_shape=[
            jax.ShapeDtypeStruct((_NP, _K), jnp.int32),
            jax.ShapeDtypeStruct((_NP, _K * _EWL), f32),
        ],
        compiler_params=pltpu.CompilerParams(
            dimension_semantics=("arbitrary",)),
    )(s, st)

    idxf = idx.reshape(_NP * _K)

    mesh = plsc.VectorSubcoreMesh(core_axis_name="c", subcore_axis_name="s")
    sum32, max32 = pl.kernel(
        _agg_body,
        out_type=[jax.ShapeDtypeStruct((_NP, _HC), f32),
                  jax.ShapeDtypeStruct((_NP, _HC), f32)],
        mesh=mesh,
        scratch_types=[
            pltpu.VMEM((_NB * _K,), jnp.int32),
            pltpu.VMEM((_NB, _K * _EWL), f32),
            pltpu.VMEM((_NB * _K, _HC), f32),
            pltpu.VMEM((_NB, _HC), f32),
            pltpu.VMEM((_NB, _HC), f32),
            pltpu.SemaphoreType.DMA,
        ],
        compiler_params=pltpu.CompilerParams(use_tc_tiling_on_sc=False),
    )(h32, idxf, ew)

    wo = p['gn_Wo']
    wox = wo[:12]
    wom32 = jnp.pad(wo[12:12 + _PROP], ((0, _HC - _PROP), (0, 0))) * (1.0 / _K)
    woM32 = jnp.pad(wo[12 + _PROP:], ((0, _HC - _PROP), (0, 0)))
    n3w0 = p['nn3_W0']
    n3wx, n3wi, n3wc = n3w0[:12], n3w0[12:18], n3w0[18:]

    ids, p4 = pl.pallas_call(
        _mlp_kernel,
        grid=(_NP // _BM,),
        in_specs=[
            pl.BlockSpec((_BM, 12), lambda i: (i, 0)),
            pl.BlockSpec((_BM, 12), lambda i: (i, 0)),
            pl.BlockSpec((_BM, _HC), lambda i: (i, 0)),
            pl.BlockSpec((_BM, _HC), lambda i: (i, 0)),
            _full((12, 64)), _full((_HC, 64)), _full((_HC, 64)), _full((1, 64)),
            _full((64, 256)), _full((1, 256)),
            _full((256, 256)), _full((1, 256)),
            _full((256, 256)), _full((1, 256)),
            _full((256, 6)), _full((1, 6)),
            _full((12, 256)), _full((6, 256)), _full((64, 256)), _full((1, 256)),
            _full((256, 256)), _full((1, 256)),
            _full((256, 256)), _full((1, 256)),
            _full((256, 6)), _full((1, 6)),
        ],
        out_specs=[
            pl.BlockSpec((_BM, 6), lambda i: (i, 0)),
            pl.BlockSpec((_BM, 6), lambda i: (i, 0)),
        ],
        out_shape=[
            jax.ShapeDtypeStruct((_NP, 6), f32),
            jax.ShapeDtypeStruct((_NP, 6), f32),
        ],
        compiler_params=pltpu.CompilerParams(
            dimension_semantics=("arbitrary",)),
    )(x1, xp, sum32, max32,
      wox, wom32, woM32, b2d('gn_bo'),
      p['nn2_W0'], b2d('nn2_b0'), p['nn2_W1'], b2d('nn2_b1'),
      p['nn2_W2'], b2d('nn2_b2'), p['nn2_W3'], b2d('nn2_b3'),
      n3wx, n3wi, n3wc, b2d('nn3_b0'),
      p['nn3_W1'], b2d('nn3_b1'), p['nn3_W2'], b2d('nn3_b2'),
      p['nn3_W3'], b2d('nn3_b3'))

    return (ids[:_N], p4[:_N], ygen_id, ygen, ycand_id, ycand)


# speedup vs baseline: 1.5977x; 1.0686x over previous
import functools

import jax
import jax.numpy as jnp
import numpy as _np
from jax import lax
from jax.experimental import pallas as pl
from jax.experimental.pallas import tpu as pltpu
from jax.experimental.pallas import tpu_sc as plsc

_N = 10000
_NP = 10240
_K = 16
_PROP = 22
_HC = 32
_BR = 128
_BE = 1024
_BM = 512
_NW = 32
_NPW = _NP // _NW
_NB = 16
_EWL = 16


def _elu(v):
    return jnp.where(v > 0, v, jnp.exp(v) - 1.0)


def _enc_kernel(x_ref, w0, b0, w1, b1, w2, b2, ws, bs, wh, bh,
                x1_ref, s_ref, h_ref):
    x = x_ref[...]
    t = _elu(jnp.dot(x, w0[...], preferred_element_type=jnp.float32) + b0[...])
    t = _elu(jnp.dot(t, w1[...], preferred_element_type=jnp.float32) + b1[...])
    x1 = jnp.dot(t, w2[...], preferred_element_type=jnp.float32) + b2[...]
    x1_ref[...] = x1
    s_ref[...] = jnp.dot(x1, ws[...], preferred_element_type=jnp.float32) + bs[...]
    h_ref[...] = jnp.dot(x1, wh[...], preferred_element_type=jnp.float32) + bh[...]


_NQ = 8
_QW = _NP // _NQ
_BIGQ = [float(_np.frombuffer(_np.uint32(0x7F000000 + _i).tobytes(),
                              _np.float32)[0]) for _i in range(_NQ)]


def _knn_kernel(s_ref, st_ref, idx_ref, ew_ref):
    s_r = s_ref[...]
    st = st_ref[...]
    sq_r = jnp.sum(s_r * s_r, axis=1, keepdims=True)
    sq_c = jnp.sum(st * st, axis=0, keepdims=True)
    m = lax.dot_general(s_r, st, (((1,), (0,)), ((), ())),
                        preferred_element_type=jnp.float32)
    d2 = sq_r + sq_c - 2.0 * m
    colq = lax.broadcasted_iota(jnp.int32, (_BR, _QW), 1)
    q = []
    for i in range(_NQ):
        di = d2[:, i * _QW:(i + 1) * _QW]
        if i == _NQ - 1:
            di = jnp.where(colq >= _N - (_NQ - 1) * _QW, _BIGQ[i], di)
        bi = (lax.bitcast_convert_type(di, jnp.int32) & ~(_NQ - 1)) | i
        q.append(lax.bitcast_convert_type(bi, jnp.float32))
    for k in range(_K):
        t = q
        while len(t) > 1:
            t = [jnp.minimum(t[j], t[j + 1]) for j in range(0, len(t), 2)]
        dm = t[0]
        mv = jnp.min(dm, axis=1, keepdims=True)
        selq = jnp.argmin(dm, axis=1, keepdims=True).astype(jnp.int32)
        qi = lax.bitcast_convert_type(mv, jnp.int32) & (_NQ - 1)
        sel = selq + _QW * qi
        idx_ref[:, k:k + 1] = sel
        wk = jnp.exp(-10.0 * jnp.maximum(mv, 0.0))
        ew_ref[:, k * _EWL:(k + 1) * _EWL] = jnp.broadcast_to(wk, (_BR, _EWL))
        for i in range(_NQ):
            q[i] = jnp.where(colq == sel - i * _QW, _BIGQ[i], q[i])


def _agg_body(h_hbm, idx_hbm, ewx_hbm, sum_hbm, max_hbm,
              idx_v, ewx_v, rows_v, sum_v, max_v, sem):
    wid = lax.axis_index("s") * 2 + lax.axis_index("c")
    node0 = wid * _NPW

    def block(b, carry):
        nb = node0 + b * _NB
        e0 = pl.multiple_of(nb * _K, _NB * _K)
        pltpu.sync_copy(idx_hbm.at[pl.ds(e0, _NB * _K)], idx_v)
        pltpu.sync_copy(ewx_hbm.at[pl.ds(nb, _NB)], ewx_v)
        pltpu.async_copy(h_hbm.at[idx_v], rows_v, sem).wait()
        for nn in range(_NB):
            for ch in range(0, _HC, 16):
                accm = jnp.zeros((16,), jnp.float32)
                accx = jnp.full((16,), -jnp.inf, jnp.float32)
                for k in range(_K):
                    v = rows_v[_K * nn + k, ch:ch + 16]
                    w = ewx_v[nn, _EWL * k:_EWL * k + 16]
                    msg = v * w
                    accm = accm + msg
                    accx = jnp.maximum(accx, msg)
                sum_v[nn, ch:ch + 16] = accm
                max_v[nn, ch:ch + 16] = accx
        pltpu.sync_copy(sum_v, sum_hbm.at[pl.ds(nb, _NB)])
        pltpu.sync_copy(max_v, max_hbm.at[pl.ds(nb, _NB)])
        return carry

    lax.fori_loop(0, _NPW // _NB, block, 0)


def _mlp_kernel(x1_ref, x_ref, sum_ref, max_ref,
                wox, wom, woM, bo,
                n2w0, n2b0, n2w1, n2b1, n2w2, n2b2, n2w3, n2b3,
                n3wx, n3wi, n3wc, n3b0, n3w1, n3b1, n3w2, n3b2, n3w3, n3b3,
                ids_ref, p4_ref):
    xc = (jnp.dot(x1_ref[...], wox[...], preferred_element_type=jnp.float32)
          + jnp.dot(sum_ref[...], wom[...], preferred_element_type=jnp.float32)
          + jnp.dot(max_ref[...], woM[...], preferred_element_type=jnp.float32)
          + bo[...])
    xc = jnp.where(xc > 0, xc, 0.01 * xc)

    t = _elu(jnp.dot(xc, n2w0[...], preferred_element_type=jnp.float32) + n2b0[...])
    t = _elu(jnp.dot(t, n2w1[...], preferred_element_type=jnp.float32) + n2b1[...])
    t = _elu(jnp.dot(t, n2w2[...], preferred_element_type=jnp.float32) + n2b2[...])
    ids = jnp.dot(t, n2w3[...], preferred_element_type=jnp.float32) + n2b3[...]
    ids_ref[...] = ids

    u = (jnp.dot(x_ref[...], n3wx[...], preferred_element_type=jnp.float32)
         + jnp.dot(ids, n3wi[...], preferred_element_type=jnp.float32)
         + jnp.dot(xc, n3wc[...], preferred_element_type=jnp.float32)
         + n3b0[...])
    u = _elu(u)
    u = _elu(jnp.dot(u, n3w1[...], preferred_element_type=jnp.float32) + n3b1[...])
    u = _elu(jnp.dot(u, n3w2[...], preferred_element_type=jnp.float32) + n3b2[...])
    p4_ref[...] = jnp.dot(u, n3w3[...], preferred_element_type=jnp.float32) + n3b3[...]


def _full(shape):
    nd = len(shape)
    return pl.BlockSpec(shape, lambda i, _nd=nd: (0,) * _nd)


def kernel(x, ygen_id, ygen, ycand_id, ycand, params):
    p = params
    f32 = jnp.float32
    xp = jnp.pad(x, ((0, _NP - _N), (0, 0)))

    def b2d(name):
        return p[name].reshape(1, -1)

    wh32 = jnp.pad(p['gn_Wh'], ((0, 0), (0, _HC - _PROP)))
    bh32 = jnp.pad(p['gn_bh'], ((0, _HC - _PROP))).reshape(1, -1)

    x1, s, h32 = pl.pallas_call(
        _enc_kernel,
        grid=(_NP // _BE,),
        in_specs=[
            pl.BlockSpec((_BE, 12), lambda i: (i, 0)),
            _full((12, 64)), _full((1, 64)),
            _full((64, 64)), _full((1, 64)),
            _full((64, 12)), _full((1, 12)),
            _full((12, 4)), _full((1, 4)),
            _full((12, _HC)), _full((1, _HC)),
        ],
        out_specs=[
            pl.BlockSpec((_BE, 12), lambda i: (i, 0)),
            pl.BlockSpec((_BE, 4), lambda i: (i, 0)),
            pl.BlockSpec((_BE, _HC), lambda i: (i, 0)),
        ],
        out_shape=[
            jax.ShapeDtypeStruct((_NP, 12), f32),
            jax.ShapeDtypeStruct((_NP, 4), f32),
            jax.ShapeDtypeStruct((_NP, _HC), f32),
        ],
        compiler_params=pltpu.CompilerParams(
            dimension_semantics=("arbitrary",)),
    )(xp, p['nn1_W0'], b2d('nn1_b0'), p['nn1_W1'], b2d('nn1_b1'),
      p['nn1_W2'], b2d('nn1_b2'), p['gn_Ws'], b2d('gn_bs'), wh32, bh32)

    st = s.T

    idx, ew = pl.pallas_call(
        _knn_kernel,
        grid=(_NP // _BR,),
        in_specs=[
            pl.BlockSpec((_BR, 4), lambda i: (i, 0)),
            _full((4, _NP)),
        ],
        out_specs=[
            pl.BlockSpec((_BR, _K), lambda i: (i, 0)),
            pl.BlockSpec((_BR, _K * _EWL), lambda i: (i, 0)),
        ],
        out_shape=[
            jax.ShapeDtypeStruct((_NP, _K), jnp.int32),
            jax.ShapeDtypeStruct((_NP, _K * _EWL), f32),
        ],
        compiler_params=pltpu.CompilerParams(
            dimension_semantics=("arbitrary",)),
    )(s, st)

    idxf = idx.reshape(_NP * _K)

    mesh = plsc.VectorSubcoreMesh(core_axis_name="c", subcore_axis_name="s")
    sum32, max32 = pl.kernel(
        _agg_body,
        out_type=[jax.ShapeDtypeStruct((_NP, _HC), f32),
                  jax.ShapeDtypeStruct((_NP, _HC), f32)],
        mesh=mesh,
        scratch_types=[
            pltpu.VMEM((_NB * _K,), jnp.int32),
            pltpu.VMEM((_NB, _K * _EWL), f32),
            pltpu.VMEM((_NB * _K, _HC), f32),
            pltpu.VMEM((_NB, _HC), f32),
            pltpu.VMEM((_NB, _HC), f32),
            pltpu.SemaphoreType.DMA,
        ],
        compiler_params=pltpu.CompilerParams(use_tc_tiling_on_sc=False),
    )(h32, idxf, ew)

    wo = p['gn_Wo']
    wox = wo[:12]
    wom32 = jnp.pad(wo[12:12 + _PROP], ((0, _HC - _PROP), (0, 0))) * (1.0 / _K)
    woM32 = jnp.pad(wo[12 + _PROP:], ((0, _HC - _PROP), (0, 0)))
    n3w0 = p['nn3_W0']
    n3wx, n3wi, n3wc = n3w0[:12], n3w0[12:18], n3w0[18:]

    ids, p4 = pl.pallas_call(
        _mlp_kernel,
        grid=(_NP // _BM,),
        in_specs=[
            pl.BlockSpec((_BM, 12), lambda i: (i, 0)),
            pl.BlockSpec((_BM, 12), lambda i: (i, 0)),
            pl.BlockSpec((_BM, _HC), lambda i: (i, 0)),
            pl.BlockSpec((_BM, _HC), lambda i: (i, 0)),
            _full((12, 64)), _full((_HC, 64)), _full((_HC, 64)), _full((1, 64)),
            _full((64, 256)), _full((1, 256)),
            _full((256, 256)), _full((1, 256)),
            _full((256, 256)), _full((1, 256)),
            _full((256, 6)), _full((1, 6)),
            _full((12, 256)), _full((6, 256)), _full((64, 256)), _full((1, 256)),
            _full((256, 256)), _full((1, 256)),
            _full((256, 256)), _full((1, 256)),
            _full((256, 6)), _full((1, 6)),
        ],
        out_specs=[
            pl.BlockSpec((_BM, 6), lambda i: (i, 0)),
            pl.BlockSpec((_BM, 6), lambda i: (i, 0)),
        ],
        out_shape=[
            jax.ShapeDtypeStruct((_NP, 6), f32),
            jax.ShapeDtypeStruct((_NP, 6), f32),
        ],
        compiler_params=pltpu.CompilerParams(
            dimension_semantics=("arbitrary",)),
    )(x1, xp, sum32, max32,
      wox, wom32, woM32, b2d('gn_bo'),
      p['nn2_W0'], b2d('nn2_b0'), p['nn2_W1'], b2d('nn2_b1'),
      p['nn2_W2'], b2d('nn2_b2'), p['nn2_W3'], b2d('nn2_b3'),
      n3wx, n3wi, n3wc, b2d('nn3_b0'),
      p['nn3_W1'], b2d('nn3_b1'), p['nn3_W2'], b2d('nn3_b2'),
      p['nn3_W3'], b2d('nn3_b3'))

    return (ids[:_N], p4[:_N], ygen_id, ygen, ycand_id, ycand)


# speedup vs baseline: 1.6200x; 1.0139x over previous
import functools

import jax
import jax.numpy as jnp
import numpy as _np
from jax import lax
from jax.experimental import pallas as pl
from jax.experimental.pallas import tpu as pltpu
from jax.experimental.pallas import tpu_sc as plsc

_N = 10000
_NP = 10240
_K = 16
_PROP = 22
_HC = 32
_BR = 128
_BE = 1024
_BM = 512
_NW = 32
_NPW = _NP // _NW
_NB = 16
_EWL = 16


def _elu(v):
    return jnp.where(v > 0, v, jnp.exp(v) - 1.0)


def _enc_kernel(x_ref, w0, b0, w1, b1, w2, b2, ws, bs, wh, bh,
                x1_ref, s_ref, h_ref):
    x = x_ref[...]
    t = _elu(jnp.dot(x, w0[...], preferred_element_type=jnp.float32) + b0[...])
    t = _elu(jnp.dot(t, w1[...], preferred_element_type=jnp.float32) + b1[...])
    x1 = jnp.dot(t, w2[...], preferred_element_type=jnp.float32) + b2[...]
    x1_ref[...] = x1
    s_ref[...] = jnp.dot(x1, ws[...], preferred_element_type=jnp.float32) + bs[...]
    h_ref[...] = jnp.dot(x1, wh[...], preferred_element_type=jnp.float32) + bh[...]


_NQ = 16
_QW = _NP // _NQ
_BIGQ = [float(_np.frombuffer(_np.uint32(0x7F000000 + _i).tobytes(),
                              _np.float32)[0]) for _i in range(_NQ)]


def _knn_kernel(s_ref, st_ref, idx_ref, ew_ref):
    s_r = s_ref[...]
    st = st_ref[...]
    sq_r = jnp.sum(s_r * s_r, axis=1, keepdims=True)
    sq_c = jnp.sum(st * st, axis=0, keepdims=True)
    m = lax.dot_general(s_r, st, (((1,), (0,)), ((), ())),
                        preferred_element_type=jnp.float32)
    d2 = sq_r + sq_c - 2.0 * m
    colq = lax.broadcasted_iota(jnp.int32, (_BR, _QW), 1)
    q = []
    for i in range(_NQ):
        di = d2[:, i * _QW:(i + 1) * _QW]
        if i == _NQ - 1:
            di = jnp.where(colq >= _N - (_NQ - 1) * _QW, _BIGQ[i], di)
        bi = (lax.bitcast_convert_type(di, jnp.int32) & ~(_NQ - 1)) | i
        q.append(lax.bitcast_convert_type(bi, jnp.float32))
    for k in range(_K):
        t = q
        while len(t) > 1:
            t = [jnp.minimum(t[j], t[j + 1]) for j in range(0, len(t), 2)]
        dm = t[0]
        mv = jnp.min(dm, axis=1, keepdims=True)
        selq = jnp.argmin(dm, axis=1, keepdims=True).astype(jnp.int32)
        qi = lax.bitcast_convert_type(mv, jnp.int32) & (_NQ - 1)
        sel = selq + _QW * qi
        idx_ref[:, k:k + 1] = sel
        wk = jnp.exp(-10.0 * jnp.maximum(mv, 0.0))
        ew_ref[:, k * _EWL:(k + 1) * _EWL] = jnp.broadcast_to(wk, (_BR, _EWL))
        for i in range(_NQ):
            q[i] = jnp.where(colq == sel - i * _QW, _BIGQ[i], q[i])


def _agg_body(h_hbm, idx_hbm, ewx_hbm, sum_hbm, max_hbm,
              idx_v, ewx_v, rows_v, sum_v, max_v, sem):
    wid = lax.axis_index("s") * 2 + lax.axis_index("c")
    node0 = wid * _NPW

    def block(b, carry):
        nb = node0 + b * _NB
        e0 = pl.multiple_of(nb * _K, _NB * _K)
        pltpu.sync_copy(idx_hbm.at[pl.ds(e0, _NB * _K)], idx_v)
        pltpu.sync_copy(ewx_hbm.at[pl.ds(nb, _NB)], ewx_v)
        pltpu.async_copy(h_hbm.at[idx_v], rows_v, sem).wait()
        for nn in range(_NB):
            for ch in range(0, _HC, 16):
                accm = jnp.zeros((16,), jnp.float32)
                accx = jnp.full((16,), -jnp.inf, jnp.float32)
                for k in range(_K):
                    v = rows_v[_K * nn + k, ch:ch + 16]
                    w = ewx_v[nn, _EWL * k:_EWL * k + 16]
                    msg = v * w
                    accm = accm + msg
                    accx = jnp.maximum(accx, msg)
                sum_v[nn, ch:ch + 16] = accm
                max_v[nn, ch:ch + 16] = accx
        pltpu.sync_copy(sum_v, sum_hbm.at[pl.ds(nb, _NB)])
        pltpu.sync_copy(max_v, max_hbm.at[pl.ds(nb, _NB)])
        return carry

    lax.fori_loop(0, _NPW // _NB, block, 0)


def _mlp_kernel(x1_ref, x_ref, sum_ref, max_ref,
                wox, wom, woM, bo,
                n2w0, n2b0, n2w1, n2b1, n2w2, n2b2, n2w3, n2b3,
                n3wx, n3wi, n3wc, n3b0, n3w1, n3b1, n3w2, n3b2, n3w3, n3b3,
                ids_ref, p4_ref):
    xc = (jnp.dot(x1_ref[...], wox[...], preferred_element_type=jnp.float32)
          + jnp.dot(sum_ref[...], wom[...], preferred_element_type=jnp.float32)
          + jnp.dot(max_ref[...], woM[...], preferred_element_type=jnp.float32)
          + bo[...])
    xc = jnp.where(xc > 0, xc, 0.01 * xc)

    t = _elu(jnp.dot(xc, n2w0[...], preferred_element_type=jnp.float32) + n2b0[...])
    t = _elu(jnp.dot(t, n2w1[...], preferred_element_type=jnp.float32) + n2b1[...])
    t = _elu(jnp.dot(t, n2w2[...], preferred_element_type=jnp.float32) + n2b2[...])
    ids = jnp.dot(t, n2w3[...], preferred_element_type=jnp.float32) + n2b3[...]
    ids_ref[...] = ids

    u = (jnp.dot(x_ref[...], n3wx[...], preferred_element_type=jnp.float32)
         + jnp.dot(ids, n3wi[...], preferred_element_type=jnp.float32)
         + jnp.dot(xc, n3wc[...], preferred_element_type=jnp.float32)
         + n3b0[...])
    u = _elu(u)
    u = _elu(jnp.dot(u, n3w1[...], preferred_element_type=jnp.float32) + n3b1[...])
    u = _elu(jnp.dot(u, n3w2[...], preferred_element_type=jnp.float32) + n3b2[...])
    p4_ref[...] = jnp.dot(u, n3w3[...], preferred_element_type=jnp.float32) + n3b3[...]


def _full(shape):
    nd = len(shape)
    return pl.BlockSpec(shape, lambda i, _nd=nd: (0,) * _nd)


def kernel(x, ygen_id, ygen, ycand_id, ycand, params):
    p = params
    f32 = jnp.float32
    xp = jnp.pad(x, ((0, _NP - _N), (0, 0)))

    def b2d(name):
        return p[name].reshape(1, -1)

    wh32 = jnp.pad(p['gn_Wh'], ((0, 0), (0, _HC - _PROP)))
    bh32 = jnp.pad(p['gn_bh'], ((0, _HC - _PROP))).reshape(1, -1)

    x1, s, h32 = pl.pallas_call(
        _enc_kernel,
        grid=(_NP // _BE,),
        in_specs=[
            pl.BlockSpec((_BE, 12), lambda i: (i, 0)),
            _full((12, 64)), _full((1, 64)),
            _full((64, 64)), _full((1, 64)),
            _full((64, 12)), _full((1, 12)),
            _full((12, 4)), _full((1, 4)),
            _full((12, _HC)), _full((1, _HC)),
        ],
        out_specs=[
            pl.BlockSpec((_BE, 12), lambda i: (i, 0)),
            pl.BlockSpec((_BE, 4), lambda i: (i, 0)),
            pl.BlockSpec((_BE, _HC), lambda i: (i, 0)),
        ],
        out_shape=[
            jax.ShapeDtypeStruct((_NP, 12), f32),
            jax.ShapeDtypeStruct((_NP, 4), f32),
            jax.ShapeDtypeStruct((_NP, _HC), f32),
        ],
        compiler_params=pltpu.CompilerParams(
            dimension_semantics=("arbitrary",)),
    )(xp, p['nn1_W0'], b2d('nn1_b0'), p['nn1_W1'], b2d('nn1_b1'),
      p['nn1_W2'], b2d('nn1_b2'), p['gn_Ws'], b2d('gn_bs'), wh32, bh32)

    st = s.T

    idx, ew = pl.pallas_call(
        _knn_kernel,
        grid=(_NP // _BR,),
        in_specs=[
            pl.BlockSpec((_BR, 4), lambda i: (i, 0)),
            _full((4, _NP)),
        ],
        out_specs=[
            pl.BlockSpec((_BR, _K), lambda i: (i, 0)),
            pl.BlockSpec((_BR, _K * _EWL), lambda i: (i, 0)),
        ],
        out_shape=[
            jax.ShapeDtypeStruct((_NP, _K), jnp.int32),
            jax.ShapeDtypeStruct((_NP, _K * _EWL), f32),
        ],
        compiler_params=pltpu.CompilerParams(
            dimension_semantics=("arbitrary",)),
    )(s, st)

    idxf = idx.reshape(_NP * _K)

    mesh = plsc.VectorSubcoreMesh(core_axis_name="c", subcore_axis_name="s")
    sum32, max32 = pl.kernel(
        _agg_body,
        out_type=[jax.ShapeDtypeStruct((_NP, _HC), f32),
                  jax.ShapeDtypeStruct((_NP, _HC), f32)],
        mesh=mesh,
        scratch_types=[
            pltpu.VMEM((_NB * _K,), jnp.int32),
            pltpu.VMEM((_NB, _K * _EWL), f32),
            pltpu.VMEM((_NB * _K, _HC), f32),
            pltpu.VMEM((_NB, _HC), f32),
            pltpu.VMEM((_NB, _HC), f32),
            pltpu.SemaphoreType.DMA,
        ],
        compiler_params=pltpu.CompilerParams(use_tc_tiling_on_sc=False),
    )(h32, idxf, ew)

    wo = p['gn_Wo']
    wox = wo[:12]
    wom32 = jnp.pad(wo[12:12 + _PROP], ((0, _HC - _PROP), (0, 0))) * (1.0 / _K)
    woM32 = jnp.pad(wo[12 + _PROP:], ((0, _HC - _PROP), (0, 0)))
    n3w0 = p['nn3_W0']
    n3wx, n3wi, n3wc = n3w0[:12], n3w0[12:18], n3w0[18:]

    ids, p4 = pl.pallas_call(
        _mlp_kernel,
        grid=(_NP // _BM,),
        in_specs=[
            pl.BlockSpec((_BM, 12), lambda i: (i, 0)),
            pl.BlockSpec((_BM, 12), lambda i: (i, 0)),
            pl.BlockSpec((_BM, _HC), lambda i: (i, 0)),
            pl.BlockSpec((_BM, _HC), lambda i: (i, 0)),
            _full((12, 64)), _full((_HC, 64)), _full((_HC, 64)), _full((1, 64)),
            _full((64, 256)), _full((1, 256)),
            _full((256, 256)), _full((1, 256)),
            _full((256, 256)), _full((1, 256)),
            _full((256, 6)), _full((1, 6)),
            _full((12, 256)), _full((6, 256)), _full((64, 256)), _full((1, 256)),
            _full((256, 256)), _full((1, 256)),
            _full((256, 256)), _full((1, 256)),
            _full((256, 6)), _full((1, 6)),
        ],
        out_specs=[
            pl.BlockSpec((_BM, 6), lambda i: (i, 0)),
            pl.BlockSpec((_BM, 6), lambda i: (i, 0)),
        ],
        out_shape=[
            jax.ShapeDtypeStruct((_NP, 6), f32),
            jax.ShapeDtypeStruct((_NP, 6), f32),
        ],
        compiler_params=pltpu.CompilerParams(
            dimension_semantics=("arbitrary",)),
    )(x1, xp, sum32, max32,
      wox, wom32, woM32, b2d('gn_bo'),
      p['nn2_W0'], b2d('nn2_b0'), p['nn2_W1'], b2d('nn2_b1'),
      p['nn2_W2'], b2d('nn2_b2'), p['nn2_W3'], b2d('nn2_b3'),
      n3wx, n3wi, n3wc, b2d('nn3_b0'),
      p['nn3_W1'], b2d('nn3_b1'), p['nn3_W2'], b2d('nn3_b2'),
      p['nn3_W3'], b2d('nn3_b3'))

    return (ids[:_N], p4[:_N], ygen_id, ygen, ycand_id, ycand)


# speedup vs baseline: 1.6917x; 1.0443x over previous
import functools

import jax
import jax.numpy as jnp
import numpy as _np
from jax import lax
from jax.experimental import pallas as pl
from jax.experimental.pallas import tpu as pltpu
from jax.experimental.pallas import tpu_sc as plsc

_N = 10000
_NP = 10240
_K = 16
_PROP = 22
_HC = 32
_BR = 256
_BE = 1024
_BM = 512
_NW = 32
_NPW = _NP // _NW
_NB = 16
_EWL = 16


def _elu(v):
    return jnp.where(v > 0, v, jnp.exp(v) - 1.0)


def _enc_kernel(x_ref, w0, b0, w1, b1, w2, b2, ws, bs, wh, bh,
                x1_ref, s_ref, h_ref):
    x = x_ref[...]
    t = _elu(jnp.dot(x, w0[...], preferred_element_type=jnp.float32) + b0[...])
    t = _elu(jnp.dot(t, w1[...], preferred_element_type=jnp.float32) + b1[...])
    x1 = jnp.dot(t, w2[...], preferred_element_type=jnp.float32) + b2[...]
    x1_ref[...] = x1
    s_ref[...] = jnp.dot(x1, ws[...], preferred_element_type=jnp.float32) + bs[...]
    h_ref[...] = jnp.dot(x1, wh[...], preferred_element_type=jnp.float32) + bh[...]


_NQ = 16
_QW = _NP // _NQ
_BIGQ = [float(_np.frombuffer(_np.uint32(0x7F000000 + _i).tobytes(),
                              _np.float32)[0]) for _i in range(_NQ)]


def _knn_kernel(s_ref, st_ref, idx_ref, ew_ref):
    s_r = s_ref[...]
    st = st_ref[...]
    sq_r = jnp.sum(s_r * s_r, axis=1, keepdims=True)
    sq_c = jnp.sum(st * st, axis=0, keepdims=True)
    m = lax.dot_general(s_r, st, (((1,), (0,)), ((), ())),
                        preferred_element_type=jnp.float32)
    d2 = sq_r + sq_c - 2.0 * m
    colq = lax.broadcasted_iota(jnp.int32, (_BR, _QW), 1)
    q = []
    for i in range(_NQ):
        di = d2[:, i * _QW:(i + 1) * _QW]
        if i == _NQ - 1:
            di = jnp.where(colq >= _N - (_NQ - 1) * _QW, _BIGQ[i], di)
        bi = (lax.bitcast_convert_type(di, jnp.int32) & ~(_NQ - 1)) | i
        q.append(lax.bitcast_convert_type(bi, jnp.float32))
    for k in range(_K):
        t = q
        while len(t) > 1:
            t = [jnp.minimum(t[j], t[j + 1]) for j in range(0, len(t), 2)]
        dm = t[0]
        mv = jnp.min(dm, axis=1, keepdims=True)
        selq = jnp.argmin(dm, axis=1, keepdims=True).astype(jnp.int32)
        qi = lax.bitcast_convert_type(mv, jnp.int32) & (_NQ - 1)
        sel = selq + _QW * qi
        idx_ref[:, k:k + 1] = sel
        wk = jnp.exp(-10.0 * jnp.maximum(mv, 0.0))
        ew_ref[:, k * _EWL:(k + 1) * _EWL] = jnp.broadcast_to(wk, (_BR, _EWL))
        for i in range(_NQ):
            q[i] = jnp.where(colq == sel - i * _QW, _BIGQ[i], q[i])


def _agg_body(h_hbm, idx_hbm, ewx_hbm, sum_hbm, max_hbm,
              idx_v, ewx_v, rows_v, sum_v, max_v, sem):
    wid = lax.axis_index("s") * 2 + lax.axis_index("c")
    node0 = wid * _NPW

    def block(b, carry):
        nb = node0 + b * _NB
        e0 = pl.multiple_of(nb * _K, _NB * _K)
        pltpu.sync_copy(idx_hbm.at[pl.ds(e0, _NB * _K)], idx_v)
        pltpu.sync_copy(ewx_hbm.at[pl.ds(nb, _NB)], ewx_v)
        pltpu.async_copy(h_hbm.at[idx_v], rows_v, sem).wait()
        for nn in range(_NB):
            for ch in range(0, _HC, 16):
                accm = jnp.zeros((16,), jnp.float32)
                accx = jnp.full((16,), -jnp.inf, jnp.float32)
                for k in range(_K):
                    v = rows_v[_K * nn + k, ch:ch + 16]
                    w = ewx_v[nn, _EWL * k:_EWL * k + 16]
                    msg = v * w
                    accm = accm + msg
                    accx = jnp.maximum(accx, msg)
                sum_v[nn, ch:ch + 16] = accm
                max_v[nn, ch:ch + 16] = accx
        pltpu.sync_copy(sum_v, sum_hbm.at[pl.ds(nb, _NB)])
        pltpu.sync_copy(max_v, max_hbm.at[pl.ds(nb, _NB)])
        return carry

    lax.fori_loop(0, _NPW // _NB, block, 0)


def _mlp_kernel(x1_ref, x_ref, sum_ref, max_ref,
                wox, wom, woM, bo,
                n2w0, n2b0, n2w1, n2b1, n2w2, n2b2, n2w3, n2b3,
                n3wx, n3wi, n3wc, n3b0, n3w1, n3b1, n3w2, n3b2, n3w3, n3b3,
                ids_ref, p4_ref):
    xc = (jnp.dot(x1_ref[...], wox[...], preferred_element_type=jnp.float32)
          + jnp.dot(sum_ref[...], wom[...], preferred_element_type=jnp.float32)
          + jnp.dot(max_ref[...], woM[...], preferred_element_type=jnp.float32)
          + bo[...])
    xc = jnp.where(xc > 0, xc, 0.01 * xc)

    t = _elu(jnp.dot(xc, n2w0[...], preferred_element_type=jnp.float32) + n2b0[...])
    t = _elu(jnp.dot(t, n2w1[...], preferred_element_type=jnp.float32) + n2b1[...])
    t = _elu(jnp.dot(t, n2w2[...], preferred_element_type=jnp.float32) + n2b2[...])
    ids = jnp.dot(t, n2w3[...], preferred_element_type=jnp.float32) + n2b3[...]
    ids_ref[...] = ids

    u = (jnp.dot(x_ref[...], n3wx[...], preferred_element_type=jnp.float32)
         + jnp.dot(ids, n3wi[...], preferred_element_type=jnp.float32)
         + jnp.dot(xc, n3wc[...], preferred_element_type=jnp.float32)
         + n3b0[...])
    u = _elu(u)
    u = _elu(jnp.dot(u, n3w1[...], preferred_element_type=jnp.float32) + n3b1[...])
    u = _elu(jnp.dot(u, n3w2[...], preferred_element_type=jnp.float32) + n3b2[...])
    p4_ref[...] = jnp.dot(u, n3w3[...], preferred_element_type=jnp.float32) + n3b3[...]


def _full(shape):
    nd = len(shape)
    return pl.BlockSpec(shape, lambda i, _nd=nd: (0,) * _nd)


def kernel(x, ygen_id, ygen, ycand_id, ycand, params):
    p = params
    f32 = jnp.float32
    xp = jnp.pad(x, ((0, _NP - _N), (0, 0)))

    def b2d(name):
        return p[name].reshape(1, -1)

    wh32 = jnp.pad(p['gn_Wh'], ((0, 0), (0, _HC - _PROP)))
    bh32 = jnp.pad(p['gn_bh'], ((0, _HC - _PROP))).reshape(1, -1)

    x1, s, h32 = pl.pallas_call(
        _enc_kernel,
        grid=(_NP // _BE,),
        in_specs=[
            pl.BlockSpec((_BE, 12), lambda i: (i, 0)),
            _full((12, 64)), _full((1, 64)),
            _full((64, 64)), _full((1, 64)),
            _full((64, 12)), _full((1, 12)),
            _full((12, 4)), _full((1, 4)),
            _full((12, _HC)), _full((1, _HC)),
        ],
        out_specs=[
            pl.BlockSpec((_BE, 12), lambda i: (i, 0)),
            pl.BlockSpec((_BE, 4), lambda i: (i, 0)),
            pl.BlockSpec((_BE, _HC), lambda i: (i, 0)),
        ],
        out_shape=[
            jax.ShapeDtypeStruct((_NP, 12), f32),
            jax.ShapeDtypeStruct((_NP, 4), f32),
            jax.ShapeDtypeStruct((_NP, _HC), f32),
        ],
        compiler_params=pltpu.CompilerParams(
            dimension_semantics=("arbitrary",)),
    )(xp, p['nn1_W0'], b2d('nn1_b0'), p['nn1_W1'], b2d('nn1_b1'),
      p['nn1_W2'], b2d('nn1_b2'), p['gn_Ws'], b2d('gn_bs'), wh32, bh32)

    st = s.T

    idx, ew = pl.pallas_call(
        _knn_kernel,
        grid=(_NP // _BR,),
        in_specs=[
            pl.BlockSpec((_BR, 4), lambda i: (i, 0)),
            _full((4, _NP)),
        ],
        out_specs=[
            pl.BlockSpec((_BR, _K), lambda i: (i, 0)),
            pl.BlockSpec((_BR, _K * _EWL), lambda i: (i, 0)),
        ],
        out_shape=[
            jax.ShapeDtypeStruct((_NP, _K), jnp.int32),
            jax.ShapeDtypeStruct((_NP, _K * _EWL), f32),
        ],
        compiler_params=pltpu.CompilerParams(
            dimension_semantics=("arbitrary",)),
    )(s, st)

    idxf = idx.reshape(_NP * _K)

    mesh = plsc.VectorSubcoreMesh(core_axis_name="c", subcore_axis_name="s")
    sum32, max32 = pl.kernel(
        _agg_body,
        out_type=[jax.ShapeDtypeStruct((_NP, _HC), f32),
                  jax.ShapeDtypeStruct((_NP, _HC), f32)],
        mesh=mesh,
        scratch_types=[
            pltpu.VMEM((_NB * _K,), jnp.int32),
            pltpu.VMEM((_NB, _K * _EWL), f32),
            pltpu.VMEM((_NB * _K, _HC), f32),
            pltpu.VMEM((_NB, _HC), f32),
            pltpu.VMEM((_NB, _HC), f32),
            pltpu.SemaphoreType.DMA,
        ],
        compiler_params=pltpu.CompilerParams(use_tc_tiling_on_sc=False),
    )(h32, idxf, ew)

    wo = p['gn_Wo']
    wox = wo[:12]
    wom32 = jnp.pad(wo[12:12 + _PROP], ((0, _HC - _PROP), (0, 0))) * (1.0 / _K)
    woM32 = jnp.pad(wo[12 + _PROP:], ((0, _HC - _PROP), (0, 0)))
    n3w0 = p['nn3_W0']
    n3wx, n3wi, n3wc = n3w0[:12], n3w0[12:18], n3w0[18:]

    ids, p4 = pl.pallas_call(
        _mlp_kernel,
        grid=(_NP // _BM,),
        in_specs=[
            pl.BlockSpec((_BM, 12), lambda i: (i, 0)),
            pl.BlockSpec((_BM, 12), lambda i: (i, 0)),
            pl.BlockSpec((_BM, _HC), lambda i: (i, 0)),
            pl.BlockSpec((_BM, _HC), lambda i: (i, 0)),
            _full((12, 64)), _full((_HC, 64)), _full((_HC, 64)), _full((1, 64)),
            _full((64, 256)), _full((1, 256)),
            _full((256, 256)), _full((1, 256)),
            _full((256, 256)), _full((1, 256)),
            _full((256, 6)), _full((1, 6)),
            _full((12, 256)), _full((6, 256)), _full((64, 256)), _full((1, 256)),
            _full((256, 256)), _full((1, 256)),
            _full((256, 256)), _full((1, 256)),
            _full((256, 6)), _full((1, 6)),
        ],
        out_specs=[
            pl.BlockSpec((_BM, 6), lambda i: (i, 0)),
            pl.BlockSpec((_BM, 6), lambda i: (i, 0)),
        ],
        out_shape=[
            jax.ShapeDtypeStruct((_NP, 6), f32),
            jax.ShapeDtypeStruct((_NP, 6), f32),
        ],
        compiler_params=pltpu.CompilerParams(
            dimension_semantics=("arbitrary",)),
    )(x1, xp, sum32, max32,
      wox, wom32, woM32, b2d('gn_bo'),
      p['nn2_W0'], b2d('nn2_b0'), p['nn2_W1'], b2d('nn2_b1'),
      p['nn2_W2'], b2d('nn2_b2'), p['nn2_W3'], b2d('nn2_b3'),
      n3wx, n3wi, n3wc, b2d('nn3_b0'),
      p['nn3_W1'], b2d('nn3_b1'), p['nn3_W2'], b2d('nn3_b2'),
      p['nn3_W3'], b2d('nn3_b3'))

    return (ids[:_N], p4[:_N], ygen_id, ygen, ycand_id, ycand)
